# baseline (speedup 1.0000x reference)
import functools

import jax
import jax.numpy as jnp
from jax import lax
from jax.experimental import pallas as pl
from jax.experimental.pallas import tpu as pltpu
from jax.experimental.pallas import tpu_sc as plsc

EMB = 64
HID = 128
P = 3
B = 4096


def _sc_gather3(user_tab, item_tab, users, pos, neg):
    info = plsc.get_sparse_core_info()
    _NC, _NS = info.num_cores, info.num_subcores
    _NW = _NC * _NS
    _BPW = B // _NW
    mesh = plsc.VectorSubcoreMesh(core_axis_name="c", subcore_axis_name="s")

    n_u = user_tab.shape[1]
    n_i = item_tab.shape[1]
    utab_flat = user_tab.reshape(P * n_u, HID)
    itab_flat = item_tab.reshape(P * n_i, HID)
    idxs = ([users + k * n_u for k in range(P)]
            + [pos + k * n_i for k in range(P)]
            + [neg + k * n_i for k in range(P)])

    @functools.partial(
        pl.kernel,
        mesh=mesh,
        out_type=[jax.ShapeDtypeStruct((B, HID), jnp.float32)] * (3 * P),
        scratch_types=[
            pltpu.VMEM((_BPW,), jnp.int32),
            pltpu.VMEM((_BPW, HID), jnp.float32),
            pltpu.SemaphoreType.DMA,
        ],
    )
    def gather3(utab, itab, *rest):
        idx_hbms = rest[:3 * P]
        outs = rest[3 * P:6 * P]
        idx_v, row_v, sem = rest[6 * P:]
        wid = lax.axis_index("s") * _NC + lax.axis_index("c")
        base = wid * _BPW
        for j, idx_hbm in enumerate(idx_hbms):
            tab = utab if j < P else itab
            pltpu.sync_copy(idx_hbm.at[pl.ds(base, _BPW)], idx_v)
            pltpu.async_copy(tab.at[idx_v], row_v, sem).wait()
            pltpu.sync_copy(row_v, outs[j].at[pl.ds(base, _BPW)])

    return gather3(utab_flat, itab_flat, *idxs)


def _max3(a, b, c):
    return jnp.maximum(jnp.maximum(a, b), c)


def _dense_body(u0_ref, u1_ref, u2_ref, p0_ref, p1_ref, p2_ref,
                n0_ref, n1_ref, n2_ref, wu_ref, wi_ref, a_ref,
                pu_ref, pi_ref, nu_ref, ni_ref):
    wu = wu_ref[...]
    wi = wi_ref[...]
    a = a_ref[...]
    proj_u = [jnp.dot(r[...], wu) for r in (u0_ref, u1_ref, u2_ref)]
    m_tmp = [jnp.dot(x, a) for x in proj_u]
    for i_refs, uo_ref, io_ref in (((p0_ref, p1_ref, p2_ref), pu_ref, pi_ref),
                                   ((n0_ref, n1_ref, n2_ref), nu_ref, ni_ref)):
        proj_i = [jnp.dot(r[...], wi) for r in i_refs]
        m = [[jnp.sum(m_tmp[p] * proj_i[q], axis=1, keepdims=True)
              for q in range(P)] for p in range(P)]
        u_logit = [_max3(m[p][0], m[p][1], m[p][2]) for p in range(P)]
        i_logit = [_max3(m[0][q], m[1][q], m[2][q]) for q in range(P)]
        um = _max3(*u_logit)
        ue = [jnp.exp(x - um) for x in u_logit]
        us = ue[0] + ue[1] + ue[2]
        uo_ref[...] = (ue[0] * proj_u[0] + ue[1] * proj_u[1]
                       + ue[2] * proj_u[2]) / us
        im = _max3(*i_logit)
        ie = [jnp.exp(x - im) for x in i_logit]
        isum = ie[0] + ie[1] + ie[2]
        io_ref[...] = (ie[0] * proj_i[0] + ie[1] * proj_i[1]
                       + ie[2] * proj_i[2]) / isum


def _dense_coattention(rows, W_u, W_i, A):
    BT = 512
    row_spec = pl.BlockSpec((BT, HID), lambda i: (i, 0))
    full = lambda shape: pl.BlockSpec(shape, lambda i: (0, 0))
    return pl.pallas_call(
        _dense_body,
        grid=(B // BT,),
        in_specs=[row_spec] * 9 + [full((HID, EMB)), full((HID, EMB)),
                                   full((EMB, EMB))],
        out_specs=[pl.BlockSpec((BT, EMB), lambda i: (i, 0))] * 4,
        out_shape=[jax.ShapeDtypeStruct((B, EMB), jnp.float32)] * 4,
    )(*rows, W_u, W_i, A)


def kernel(users, pos_items, neg_items, multi_user_embed, multi_item_embed,
           W_u, W_i, A):
    t_utab = jnp.transpose(multi_user_embed, (1, 0, 2))
    t_itab = jnp.transpose(multi_item_embed, (1, 0, 2))
    rows = _sc_gather3(
        t_utab, t_itab,
        users.astype(jnp.int32), pos_items.astype(jnp.int32),
        neg_items.astype(jnp.int32))
    pu, pi, nu, ni = _dense_coattention(rows, W_u, W_i, A)
    return (pu, pi, nu, ni)


# speedup vs baseline: 1.3128x; 1.3128x over previous
import functools

import jax
import jax.numpy as jnp
from jax import lax
from jax.experimental import pallas as pl
from jax.experimental.pallas import tpu as pltpu
from jax.experimental.pallas import tpu_sc as plsc

EMB = 64
HID = 128
P = 3
B = 4096


def _sc_gather3(user_tab, item_tab, users, pos, neg):
    info = plsc.get_sparse_core_info()
    _NC, _NS = info.num_cores, info.num_subcores
    _NW = _NC * _NS
    _BPW = B // _NW
    mesh = plsc.VectorSubcoreMesh(core_axis_name="c", subcore_axis_name="s")

    n_u = user_tab.shape[1]
    n_i = item_tab.shape[1]
    utab_flat = user_tab.reshape(P * n_u, HID)
    itab_flat = item_tab.reshape(P * n_i, HID)
    idxs = ([users + k * n_u for k in range(P)]
            + [pos + k * n_i for k in range(P)]
            + [neg + k * n_i for k in range(P)])

    @functools.partial(
        pl.kernel,
        mesh=mesh,
        out_type=[jax.ShapeDtypeStruct((B, HID), jnp.float32)] * (3 * P),
        scratch_types=[
            pltpu.VMEM((_BPW,), jnp.int32),
            pltpu.VMEM((_BPW, HID), jnp.float32),
            pltpu.SemaphoreType.DMA,
        ],
    )
    def gather3(utab, itab, *rest):
        idx_hbms = rest[:3 * P]
        outs = rest[3 * P:6 * P]
        idx_v, row_v, sem = rest[6 * P:]
        wid = lax.axis_index("s") * _NC + lax.axis_index("c")
        base = wid * _BPW
        for j, idx_hbm in enumerate(idx_hbms):
            tab = utab if j < P else itab
            pltpu.sync_copy(idx_hbm.at[pl.ds(base, _BPW)], idx_v)
            pltpu.async_copy(tab.at[idx_v], row_v, sem).wait()
            pltpu.sync_copy(row_v, outs[j].at[pl.ds(base, _BPW)])

    return gather3(utab_flat, itab_flat, *idxs)


def _max3(a, b, c):
    return jnp.maximum(jnp.maximum(a, b), c)


def _dense_body(u0_ref, u1_ref, u2_ref, p0_ref, p1_ref, p2_ref,
                n0_ref, n1_ref, n2_ref, wu_ref, wi_ref, a_ref,
                pu_ref, pi_ref, nu_ref, ni_ref):
    wu = wu_ref[...]
    wi = wi_ref[...]
    a = a_ref[...]

    def proj_t(r, w):
        return lax.dot_general(w, r[...], (((0,), (1,)), ((), ())))

    proj_u = [proj_t(r, wu) for r in (u0_ref, u1_ref, u2_ref)]
    m_tmp = [lax.dot_general(a, x, (((0,), (0,)), ((), ()))) for x in proj_u]
    for i_refs, uo_ref, io_ref in (((p0_ref, p1_ref, p2_ref), pu_ref, pi_ref),
                                   ((n0_ref, n1_ref, n2_ref), nu_ref, ni_ref)):
        proj_i = [proj_t(r, wi) for r in i_refs]
        m = [[jnp.sum(m_tmp[p] * proj_i[q], axis=0, keepdims=True)
              for q in range(P)] for p in range(P)]
        u_logit = [_max3(m[p][0], m[p][1], m[p][2]) for p in range(P)]
        i_logit = [_max3(m[0][q], m[1][q], m[2][q]) for q in range(P)]
        um = _max3(*u_logit)
        ue = [jnp.exp(x - um) for x in u_logit]
        us = ue[0] + ue[1] + ue[2]
        uo_ref[...] = (ue[0] * proj_u[0] + ue[1] * proj_u[1]
                       + ue[2] * proj_u[2]) / us
        im = _max3(*i_logit)
        ie = [jnp.exp(x - im) for x in i_logit]
        isum = ie[0] + ie[1] + ie[2]
        io_ref[...] = (ie[0] * proj_i[0] + ie[1] * proj_i[1]
                       + ie[2] * proj_i[2]) / isum


def _dense_coattention(rows, W_u, W_i, A):
    BT = 512
    row_spec = pl.BlockSpec((BT, HID), lambda i: (i, 0))
    full = lambda shape: pl.BlockSpec(shape, lambda i: (0, 0))
    return pl.pallas_call(
        _dense_body,
        grid=(B // BT,),
        in_specs=[row_spec] * 9 + [full((HID, EMB)), full((HID, EMB)),
                                   full((EMB, EMB))],
        out_specs=[pl.BlockSpec((EMB, BT), lambda i: (0, i))] * 4,
        out_shape=[jax.ShapeDtypeStruct((EMB, B), jnp.float32)] * 4,
    )(*rows, W_u, W_i, A)


def kernel(users, pos_items, neg_items, multi_user_embed, multi_item_embed,
           W_u, W_i, A):
    t_utab = jnp.transpose(multi_user_embed, (1, 0, 2))
    t_itab = jnp.transpose(multi_item_embed, (1, 0, 2))
    rows = _sc_gather3(
        t_utab, t_itab,
        users.astype(jnp.int32), pos_items.astype(jnp.int32),
        neg_items.astype(jnp.int32))
    pu, pi, nu, ni = _dense_coattention(rows, W_u, W_i, A)
    return (pu.T, pi.T, nu.T, ni.T)


# speedup vs baseline: 1.4663x; 1.1170x over previous
import functools

import jax
import jax.numpy as jnp
from jax import lax
from jax.experimental import pallas as pl
from jax.experimental.pallas import tpu as pltpu
from jax.experimental.pallas import tpu_sc as plsc

EMB = 64
HID = 128
P = 3
B = 4096


def _sc_gather3(user_tab, item_tab, users, pos, neg):
    info = plsc.get_sparse_core_info()
    _NC, _NS = info.num_cores, info.num_subcores
    _NW = _NC * _NS
    _BPW = B // _NW
    mesh = plsc.VectorSubcoreMesh(core_axis_name="c", subcore_axis_name="s")

    n_u = user_tab.shape[1]
    n_i = item_tab.shape[1]
    utab_flat = user_tab.reshape(P * n_u, HID)
    itab_flat = item_tab.reshape(P * n_i, HID)
    idxs = ([users + k * n_u for k in range(P)]
            + [pos + k * n_i for k in range(P)]
            + [neg + k * n_i for k in range(P)])

    @functools.partial(
        pl.kernel,
        mesh=mesh,
        out_type=[jax.ShapeDtypeStruct((B, HID), jnp.float32)] * (3 * P),
        scratch_types=[
            pltpu.VMEM((3 * P, _BPW), jnp.int32),
            pltpu.VMEM((_BPW, HID), jnp.float32),
            pltpu.VMEM((_BPW, HID), jnp.float32),
            pltpu.SemaphoreType.DMA,
            pltpu.SemaphoreType.DMA,
            pltpu.SemaphoreType.DMA,
        ],
    )
    def gather3(utab, itab, *rest):
        idx_hbms = rest[:3 * P]
        outs = rest[3 * P:6 * P]
        idx_all, buf0, buf1, isem, gsem, osem = rest[6 * P:]
        wid = lax.axis_index("s") * _NC + lax.axis_index("c")
        base = wid * _BPW
        nj = 3 * P
        idx_copies = [
            pltpu.make_async_copy(idx_hbms[j].at[pl.ds(base, _BPW)],
                                  idx_all.at[j], isem)
            for j in range(nj)
        ]
        for c in idx_copies:
            c.start()
        for c in idx_copies:
            c.wait()
        bufs = [buf0, buf1]
        tabs = [utab] * P + [itab] * (2 * P)
        gathers = [
            pltpu.make_async_copy(tabs[j].at[idx_all.at[j]], bufs[j % 2], gsem)
            for j in range(nj)
        ]
        outcopies = [
            pltpu.make_async_copy(bufs[j % 2], outs[j].at[pl.ds(base, _BPW)],
                                  osem)
            for j in range(nj)
        ]
        gathers[0].start()
        for j in range(nj):
            gathers[j].wait()
            outcopies[j].start()
            if j + 1 < nj:
                if j >= 1:
                    outcopies[j - 1].wait()
                gathers[j + 1].start()
        outcopies[nj - 2].wait()
        outcopies[nj - 1].wait()

    return gather3(utab_flat, itab_flat, *idxs)


def _max3(a, b, c):
    return jnp.maximum(jnp.maximum(a, b), c)


def _dense_body(u0_ref, u1_ref, u2_ref, p0_ref, p1_ref, p2_ref,
                n0_ref, n1_ref, n2_ref, wu_ref, wi_ref, a_ref,
                pu_ref, pi_ref, nu_ref, ni_ref):
    wu = wu_ref[...]
    wi = wi_ref[...]
    a = a_ref[...]

    def proj_t(r, w):
        return lax.dot_general(w, r[...], (((0,), (1,)), ((), ())))

    proj_u = [proj_t(r, wu) for r in (u0_ref, u1_ref, u2_ref)]
    m_tmp = [lax.dot_general(a, x, (((0,), (0,)), ((), ()))) for x in proj_u]
    for i_refs, uo_ref, io_ref in (((p0_ref, p1_ref, p2_ref), pu_ref, pi_ref),
                                   ((n0_ref, n1_ref, n2_ref), nu_ref, ni_ref)):
        proj_i = [proj_t(r, wi) for r in i_refs]
        m = [[jnp.sum(m_tmp[p] * proj_i[q], axis=0, keepdims=True)
              for q in range(P)] for p in range(P)]
        u_logit = [_max3(m[p][0], m[p][1], m[p][2]) for p in range(P)]
        i_logit = [_max3(m[0][q], m[1][q], m[2][q]) for q in range(P)]
        um = _max3(*u_logit)
        ue = [jnp.exp(x - um) for x in u_logit]
        us = ue[0] + ue[1] + ue[2]
        uo_ref[...] = (ue[0] * proj_u[0] + ue[1] * proj_u[1]
                       + ue[2] * proj_u[2]) / us
        im = _max3(*i_logit)
        ie = [jnp.exp(x - im) for x in i_logit]
        isum = ie[0] + ie[1] + ie[2]
        io_ref[...] = (ie[0] * proj_i[0] + ie[1] * proj_i[1]
                       + ie[2] * proj_i[2]) / isum


def _dense_coattention(rows, W_u, W_i, A):
    BT = 512
    row_spec = pl.BlockSpec((BT, HID), lambda i: (i, 0))
    full = lambda shape: pl.BlockSpec(shape, lambda i: (0, 0))
    return pl.pallas_call(
        _dense_body,
        grid=(B // BT,),
        in_specs=[row_spec] * 9 + [full((HID, EMB)), full((HID, EMB)),
                                   full((EMB, EMB))],
        out_specs=[pl.BlockSpec((EMB, BT), lambda i: (0, i))] * 4,
        out_shape=[jax.ShapeDtypeStruct((EMB, B), jnp.float32)] * 4,
    )(*rows, W_u, W_i, A)


def kernel(users, pos_items, neg_items, multi_user_embed, multi_item_embed,
           W_u, W_i, A):
    t_utab = jnp.transpose(multi_user_embed, (1, 0, 2))
    t_itab = jnp.transpose(multi_item_embed, (1, 0, 2))
    rows = _sc_gather3(
        t_utab, t_itab,
        users.astype(jnp.int32), pos_items.astype(jnp.int32),
        neg_items.astype(jnp.int32))
    pu, pi, nu, ni = _dense_coattention(rows, W_u, W_i, A)
    return (pu.T, pi.T, nu.T, ni.T)


# speedup vs baseline: 1.5970x; 1.0891x over previous
import functools

import jax
import jax.numpy as jnp
from jax import lax
from jax.experimental import pallas as pl
from jax.experimental.pallas import tpu as pltpu
from jax.experimental.pallas import tpu_sc as plsc

EMB = 64
HID = 128
P = 3
B = 4096


def _sc_gather3(user_tab, item_tab, users, pos, neg):
    info = plsc.get_sparse_core_info()
    _NC, _NS = info.num_cores, info.num_subcores
    _NW = _NC * _NS
    _BPW = B // _NW
    mesh = plsc.VectorSubcoreMesh(core_axis_name="c", subcore_axis_name="s")

    n_u = user_tab.shape[1]
    n_i = item_tab.shape[1]
    utab_flat = user_tab.reshape(P * n_u, HID)
    itab_flat = item_tab.reshape(P * n_i, HID)
    nj = 3 * P
    NBUF = 4

    @functools.partial(
        pl.kernel,
        mesh=mesh,
        out_type=[jax.ShapeDtypeStruct((B, HID), jnp.float32)] * nj,
        scratch_types=[
            pltpu.VMEM((nj, _BPW), jnp.int32),
            *([pltpu.VMEM((_BPW, HID), jnp.float32)] * NBUF),
            pltpu.SemaphoreType.DMA,
            pltpu.SemaphoreType.DMA,
            pltpu.SemaphoreType.DMA,
        ],
    )
    def gather3(utab, itab, u_idx, p_idx, n_idx, *rest):
        outs = rest[:nj]
        idx_all = rest[nj]
        bufs = list(rest[nj + 1:nj + 1 + NBUF])
        isem, gsem, osem = rest[nj + 1 + NBUF:]
        wid = lax.axis_index("s") * _NC + lax.axis_index("c")
        base = wid * _BPW
        idx_copies = [
            pltpu.make_async_copy(idx_hbm.at[pl.ds(base, _BPW)],
                                  idx_all.at[3 * g], isem)
            for g, idx_hbm in enumerate((u_idx, p_idx, n_idx))
        ]
        for c in idx_copies:
            c.start()
        for c in idx_copies:
            c.wait()
        for g, n_tab in enumerate((n_u, n_i, n_i)):
            for k in range(1, P):
                for m in range(_BPW // 16):
                    sl = pl.ds(m * 16, 16)
                    idx_all[3 * g + k, sl] = idx_all[3 * g, sl] + k * n_tab
        tabs = [utab] * P + [itab] * (2 * P)
        gathers = [
            pltpu.make_async_copy(tabs[j].at[idx_all.at[j]],
                                  bufs[j % NBUF], gsem)
            for j in range(nj)
        ]
        outcopies = [
            pltpu.make_async_copy(bufs[j % NBUF],
                                  outs[j].at[pl.ds(base, _BPW)], osem)
            for j in range(nj)
        ]
        for j in range(NBUF - 1):
            gathers[j].start()
        waited = [False] * nj
        for j in range(nj):
            gathers[j].wait()
            outcopies[j].start()
            if j + NBUF - 1 < nj:
                if j >= 1:
                    outcopies[j - 1].wait()
                    waited[j - 1] = True
                gathers[j + NBUF - 1].start()
        for j in range(nj):
            if not waited[j]:
                outcopies[j].wait()

    return gather3(utab_flat, itab_flat,
                   users.astype(jnp.int32), pos.astype(jnp.int32),
                   neg.astype(jnp.int32))


def _max3(a, b, c):
    return jnp.maximum(jnp.maximum(a, b), c)


def _dense_body(u0_ref, u1_ref, u2_ref, p0_ref, p1_ref, p2_ref,
                n0_ref, n1_ref, n2_ref, wu_ref, wi_ref, a_ref,
                pu_ref, pi_ref, nu_ref, ni_ref):
    wu = wu_ref[...]
    wi = wi_ref[...]
    a = a_ref[...]

    def proj_t(r, w):
        return lax.dot_general(w, r[...], (((0,), (1,)), ((), ())))

    proj_u = [proj_t(r, wu) for r in (u0_ref, u1_ref, u2_ref)]
    m_tmp = [lax.dot_general(a, x, (((0,), (0,)), ((), ()))) for x in proj_u]
    for i_refs, uo_ref, io_ref in (((p0_ref, p1_ref, p2_ref), pu_ref, pi_ref),
                                   ((n0_ref, n1_ref, n2_ref), nu_ref, ni_ref)):
        proj_i = [proj_t(r, wi) for r in i_refs]
        m = [[jnp.sum(m_tmp[p] * proj_i[q], axis=0, keepdims=True)
              for q in range(P)] for p in range(P)]
        u_logit = [_max3(m[p][0], m[p][1], m[p][2]) for p in range(P)]
        i_logit = [_max3(m[0][q], m[1][q], m[2][q]) for q in range(P)]
        um = _max3(*u_logit)
        ue = [jnp.exp(x - um) for x in u_logit]
        us = ue[0] + ue[1] + ue[2]
        uo_ref[...] = (ue[0] * proj_u[0] + ue[1] * proj_u[1]
                       + ue[2] * proj_u[2]) / us
        im = _max3(*i_logit)
        ie = [jnp.exp(x - im) for x in i_logit]
        isum = ie[0] + ie[1] + ie[2]
        io_ref[...] = (ie[0] * proj_i[0] + ie[1] * proj_i[1]
                       + ie[2] * proj_i[2]) / isum


def _dense_coattention(rows, W_u, W_i, A):
    BT = 512
    row_spec = pl.BlockSpec((BT, HID), lambda i: (i, 0))
    full = lambda shape: pl.BlockSpec(shape, lambda i: (0, 0))
    return pl.pallas_call(
        _dense_body,
        grid=(B // BT,),
        in_specs=[row_spec] * 9 + [full((HID, EMB)), full((HID, EMB)),
                                   full((EMB, EMB))],
        out_specs=[pl.BlockSpec((EMB, BT), lambda i: (0, i))] * 4,
        out_shape=[jax.ShapeDtypeStruct((EMB, B), jnp.float32)] * 4,
    )(*rows, W_u, W_i, A)


def kernel(users, pos_items, neg_items, multi_user_embed, multi_item_embed,
           W_u, W_i, A):
    t_utab = jnp.transpose(multi_user_embed, (1, 0, 2))
    t_itab = jnp.transpose(multi_item_embed, (1, 0, 2))
    rows = _sc_gather3(
        t_utab, t_itab,
        users.astype(jnp.int32), pos_items.astype(jnp.int32),
        neg_items.astype(jnp.int32))
    pu, pi, nu, ni = _dense_coattention(rows, W_u, W_i, A)
    return (pu.T, pi.T, nu.T, ni.T)


# speedup vs baseline: 1.6725x; 1.0473x over previous
import functools

import jax
import jax.numpy as jnp
from jax import lax
from jax.experimental import pallas as pl
from jax.experimental.pallas import tpu as pltpu
from jax.experimental.pallas import tpu_sc as plsc

EMB = 64
HID = 128
P = 3
B = 4096


def _sc_gather3(user_tab, item_tab, users, pos, neg):
    info = plsc.get_sparse_core_info()
    _NC, _NS = info.num_cores, info.num_subcores
    _NW = _NC * _NS
    _BPW = B // _NW
    mesh = plsc.VectorSubcoreMesh(core_axis_name="c", subcore_axis_name="s")

    n_u = user_tab.shape[1]
    n_i = item_tab.shape[1]
    utab_flat = user_tab.reshape(P * n_u, HID)
    itab_flat = item_tab.reshape(P * n_i, HID)
    nj = 3 * P
    NBUF = 4

    @functools.partial(
        pl.kernel,
        mesh=mesh,
        out_type=[jax.ShapeDtypeStruct((B, HID), jnp.float32)] * nj,
        scratch_types=[
            pltpu.VMEM((nj, _BPW), jnp.int32),
            *([pltpu.VMEM((_BPW, HID), jnp.float32)] * NBUF),
            pltpu.SemaphoreType.DMA,
            pltpu.SemaphoreType.DMA,
            pltpu.SemaphoreType.DMA,
        ],
    )
    def gather3(utab, itab, u_idx, p_idx, n_idx, *rest):
        outs = rest[:nj]
        idx_all = rest[nj]
        bufs = list(rest[nj + 1:nj + 1 + NBUF])
        isem, gsem, osem = rest[nj + 1 + NBUF:]
        wid = lax.axis_index("s") * _NC + lax.axis_index("c")
        base = wid * _BPW
        idx_copies = [
            pltpu.make_async_copy(idx_hbm.at[pl.ds(base, _BPW)],
                                  idx_all.at[3 * g], isem)
            for g, idx_hbm in enumerate((u_idx, p_idx, n_idx))
        ]
        for c in idx_copies:
            c.start()
        for c in idx_copies:
            c.wait()
        for g, n_tab in enumerate((n_u, n_i, n_i)):
            for k in range(1, P):
                for m in range(_BPW // 16):
                    sl = pl.ds(m * 16, 16)
                    idx_all[3 * g + k, sl] = idx_all[3 * g, sl] + k * n_tab
        tabs = [utab] * P + [itab] * (2 * P)
        gathers = [
            pltpu.make_async_copy(tabs[j].at[idx_all.at[j]],
                                  bufs[j % NBUF], gsem)
            for j in range(nj)
        ]
        outcopies = [
            pltpu.make_async_copy(bufs[j % NBUF],
                                  outs[j].at[pl.ds(base, _BPW)], osem)
            for j in range(nj)
        ]
        for j in range(NBUF - 1):
            gathers[j].start()
        waited = [False] * nj
        for j in range(nj):
            gathers[j].wait()
            outcopies[j].start()
            if j + NBUF - 1 < nj:
                if j >= 1:
                    outcopies[j - 1].wait()
                    waited[j - 1] = True
                gathers[j + NBUF - 1].start()
        for j in range(nj):
            if not waited[j]:
                outcopies[j].wait()

    return gather3(utab_flat, itab_flat,
                   users.astype(jnp.int32), pos.astype(jnp.int32),
                   neg.astype(jnp.int32))


def _max3(a, b, c):
    return jnp.maximum(jnp.maximum(a, b), c)


def _dense_body(u0_ref, u1_ref, u2_ref, p0_ref, p1_ref, p2_ref,
                n0_ref, n1_ref, n2_ref, wu_ref, wi_ref, a_ref,
                pu_ref, pi_ref, nu_ref, ni_ref):
    wu = wu_ref[...]
    wi = wi_ref[...]
    a = a_ref[...]

    def proj_t(r, w):
        return lax.dot_general(w, r[...], (((0,), (1,)), ((), ())))

    proj_u = [proj_t(r, wu) for r in (u0_ref, u1_ref, u2_ref)]
    m_tmp = [lax.dot_general(a, x, (((0,), (0,)), ((), ()))) for x in proj_u]
    for i_refs, uo_ref, io_ref in (((p0_ref, p1_ref, p2_ref), pu_ref, pi_ref),
                                   ((n0_ref, n1_ref, n2_ref), nu_ref, ni_ref)):
        proj_i = [proj_t(r, wi) for r in i_refs]
        m = [[jnp.sum(m_tmp[p] * proj_i[q], axis=0, keepdims=True)
              for q in range(P)] for p in range(P)]
        u_logit = [_max3(m[p][0], m[p][1], m[p][2]) for p in range(P)]
        i_logit = [_max3(m[0][q], m[1][q], m[2][q]) for q in range(P)]
        um = _max3(*u_logit)
        ue = [jnp.exp(x - um) for x in u_logit]
        us = ue[0] + ue[1] + ue[2]
        uo_ref[...] = (ue[0] * proj_u[0] + ue[1] * proj_u[1]
                       + ue[2] * proj_u[2]) / us
        im = _max3(*i_logit)
        ie = [jnp.exp(x - im) for x in i_logit]
        isum = ie[0] + ie[1] + ie[2]
        io_ref[...] = (ie[0] * proj_i[0] + ie[1] * proj_i[1]
                       + ie[2] * proj_i[2]) / isum


def _dense_coattention(rows, W_u, W_i, A):
    BT = 1024
    row_spec = pl.BlockSpec((BT, HID), lambda i: (i, 0))
    full = lambda shape: pl.BlockSpec(shape, lambda i: (0, 0))
    return pl.pallas_call(
        _dense_body,
        grid=(B // BT,),
        in_specs=[row_spec] * 9 + [full((HID, EMB)), full((HID, EMB)),
                                   full((EMB, EMB))],
        out_specs=[pl.BlockSpec((EMB, BT), lambda i: (0, i))] * 4,
        out_shape=[jax.ShapeDtypeStruct((EMB, B), jnp.float32)] * 4,
    )(*rows, W_u, W_i, A)


def kernel(users, pos_items, neg_items, multi_user_embed, multi_item_embed,
           W_u, W_i, A):
    t_utab = jnp.transpose(multi_user_embed, (1, 0, 2))
    t_itab = jnp.transpose(multi_item_embed, (1, 0, 2))
    rows = _sc_gather3(
        t_utab, t_itab,
        users.astype(jnp.int32), pos_items.astype(jnp.int32),
        neg_items.astype(jnp.int32))
    pu, pi, nu, ni = _dense_coattention(rows, W_u, W_i, A)
    return (pu.T, pi.T, nu.T, ni.T)
